# baseline (speedup 1.0000x reference)
import functools

import numpy as np
import jax
import jax.numpy as jnp
from jax import lax
from jax.experimental import pallas as pl
from jax.experimental.pallas import tpu as pltpu
from jax.experimental.pallas import tpu_sc as plsc

N_LEVELS = 8
F = 8
TSIZE = 1 << 18
HMASK = TSIZE - 1
BASE_RES = 32
MAX_RES = 8192
NUM_BASIS = 4
HIDDEN = 64
B = 262144
OUT_DIM = 6
_scale = np.exp2(np.log2(MAX_RES / BASE_RES) / (N_LEVELS - 1))
RES = [int(np.floor(BASE_RES * _scale ** l)) for l in range(N_LEVELS)]
DENSE = [(r + 1) ** 3 <= TSIZE for r in RES]
PRIME1 = int(np.int32(np.uint32(2654435761).astype(np.int32)))
PRIME2 = int(np.int32(np.uint32(805459861).astype(np.int32)))

_NC = 2
_NS = 16
_NW = _NC * _NS
_PB = B // _NW
_C = 128
_NCHUNK = _PB // _C
_G = _C // 16
_NIDX = N_LEVELS * 8

_NEB = TSIZE // 128
_BEB = 8
_PCH = N_LEVELS * _NEB // _NW


def _sc_tabprep(tabs, bb):
    mesh = plsc.VectorSubcoreMesh(core_axis_name="c", subcore_axis_name="s")

    @functools.partial(
        pl.kernel,
        out_type=jax.ShapeDtypeStruct((N_LEVELS, TSIZE, F), jnp.float32),
        mesh=mesh,
        compiler_params=pltpu.CompilerParams(needs_layout_passes=False,
                                             use_tc_tiling_on_sc=False),
        scratch_types=[
            pltpu.VMEM((_BEB, F, 128), jnp.float32),
            pltpu.VMEM((_BEB * 128, F), jnp.float32),
            pltpu.VMEM((NUM_BASIS * 16,), jnp.float32),
            pltpu.SemaphoreType.DMA,
        ],
    )
    def prep(tabs_h, bb_h, out_h, tin, tout, bbv, sem):
        wid = lax.axis_index("s") * _NC + lax.axis_index("c")
        cid0 = wid * _PCH
        lvl = cid0 // _NEB
        eb0 = cid0 % _NEB

        pltpu.sync_copy(bb_h, bbv)

        def batch(bi, carry):
            eb = eb0 + bi * _BEB
            pltpu.sync_copy(tabs_h.at[lvl, pl.ds(eb, _BEB)], tin)
            bvs = [bbv[pl.ds(16 * j, 16)] for j in range(NUM_BASIS)]
            for e in range(_BEB):
                for g in range(8):
                    fv = [tin[e, f, pl.ds(g * 16, 16)] for f in range(F)]
                    y0 = (bvs[0] * fv[0] + bvs[1] * fv[2]
                          + bvs[2] * fv[4] + bvs[3] * fv[6])
                    y1 = (bvs[0] * fv[1] + bvs[1] * fv[3]
                          + bvs[2] * fv[5] + bvs[3] * fv[7])
                    ridx = (jnp.int32(e * 128 + g * 16)
                            + lax.iota(jnp.int32, 16))
                    plsc.store_scatter(
                        tout, [ridx, jnp.full((16,), 0, jnp.int32)], y0)
                    plsc.store_scatter(
                        tout, [ridx, jnp.full((16,), 1, jnp.int32)], y1)
            pltpu.sync_copy(tout, out_h.at[lvl, pl.ds(eb * 128, _BEB * 128)])
            return carry

        lax.fori_loop(0, _PCH // _BEB, batch, 0, unroll=False)

    return prep(tabs, bb)


def _sc_encode(xq, tab):
    mesh = plsc.VectorSubcoreMesh(core_axis_name="c", subcore_axis_name="s")

    @functools.partial(
        pl.kernel,
        out_type=jax.ShapeDtypeStruct((2 * N_LEVELS, B), jnp.float32),
        mesh=mesh,
        compiler_params=pltpu.CompilerParams(needs_layout_passes=False,
                                             use_tc_tiling_on_sc=False),
        scratch_types=[
            pltpu.VMEM((4, 128), jnp.float32),
            pltpu.VMEM((N_LEVELS, 8 * _C), jnp.int32),
            pltpu.VMEM((_NIDX, 128), jnp.float32),
            pltpu.VMEM((2 * 8 * _C, F), jnp.float32),
            pltpu.VMEM((2 * N_LEVELS, _C), jnp.float32),
            pltpu.SemaphoreType.DMA,
            pltpu.SemaphoreType.DMA,
        ],
    )
    def enc(xq_h, tab_h, out_h, xtv, idxb, wb, rows, hb, sem0, sem1):
        wid = lax.axis_index("s") * _NC + lax.axis_index("c")
        tbase = wid * _PB

        def chunk_body(ci, carry):
            cbase = tbase + ci * _C
            pltpu.sync_copy(xq_h.at[wid * (_PB // 128) + ci], xtv)

            def idx_body(g, c2):
                s = g * 16
                x = xtv[0, pl.ds(s, 16)]
                y = xtv[1, pl.ds(s, 16)]
                z = xtv[2, pl.ds(s, 16)]
                for l in range(N_LEVELS):
                    res = RES[l]
                    px = x * np.float32(res)
                    py = y * np.float32(res)
                    pz = z * np.float32(res)
                    ix = px.astype(jnp.int32)
                    iy = py.astype(jnp.int32)
                    iz = pz.astype(jnp.int32)
                    fx = px - ix.astype(jnp.float32)
                    fy = py - iy.astype(jnp.float32)
                    fz = pz - iz.astype(jnp.float32)
                    if DENSE[l]:
                        st = res + 1
                        a0 = ix
                        a1 = a0 + 1
                        b0 = iy * jnp.int32(st)
                        b1 = b0 + jnp.int32(st)
                        c0 = iz * jnp.int32(st * st)
                        c1 = c0 + jnp.int32(st * st)
                        idx8 = [a + bc + cc
                                for a in (a0, a1) for bc in (b0, b1)
                                for cc in (c0, c1)]
                    else:
                        a0 = ix
                        a1 = ix + 1
                        b0 = iy * jnp.int32(PRIME1)
                        b1 = b0 + jnp.int32(PRIME1)
                        c0 = iz * jnp.int32(PRIME2)
                        c1 = c0 + jnp.int32(PRIME2)
                        idx8 = [(a ^ bc ^ cc) & jnp.int32(HMASK)
                                for a in (a0, a1) for bc in (b0, b1)
                                for cc in (c0, c1)]
                    ux = 1.0 - fx
                    uy = 1.0 - fy
                    uz = 1.0 - fz
                    wxy = [ux * uy, ux * fy, fx * uy, fx * fy]
                    for k in range(8):
                        wv = wxy[k >> 1] * (fz if (k & 1) else uz)
                        idxb[l, pl.ds(k * _C + s, 16)] = idx8[k]
                        wb[l * 8 + k, pl.ds(s, 16)] = wv
                return c2

            lax.fori_loop(0, _G, idx_body, 0, unroll=False)

            sems = (sem0, sem1)

            def fire(l):
                pb = l % 2
                return [pltpu.async_copy(
                    tab_h.at[l].at[idxb.at[l]],
                    rows.at[pl.ds(pb * 8 * _C, 8 * _C)], sems[pb])]

            def acc_level(l):
                pb = l % 2

                def acc_body(g, c2):
                    s = g * 16
                    acc = [None, None]
                    for k in range(8):
                        wv = wb[l * 8 + k, pl.ds(s, 16)]
                        ridx = (jnp.int32(pb * 8 * _C + k * _C) + s
                                + lax.iota(jnp.int32, 16))
                        for f in range(2):
                            v = plsc.load_gather(
                                rows, [ridx, jnp.full((16,), f, jnp.int32)])
                            vv = wv * v
                            acc[f] = vv if acc[f] is None else acc[f] + vv
                    hb[2 * l, pl.ds(s, 16)] = acc[0]
                    hb[2 * l + 1, pl.ds(s, 16)] = acc[1]
                    return c2

                lax.fori_loop(0, _G, acc_body, 0, unroll=False)

            inflight = fire(0)
            for l in range(N_LEVELS):
                nxt = fire(l + 1) if l + 1 < N_LEVELS else []
                for cp in inflight:
                    cp.wait()
                inflight = nxt
                acc_level(l)

            pltpu.sync_copy(hb, out_h.at[:, pl.ds(cbase, _C)])
            return carry

        lax.fori_loop(0, _NCHUNK, chunk_body, 0, unroll=False)

    return enc(xq, tab)


_TB = 2048


def _tc_mlp(ht, W0, W1, W2):
    def body(ht_ref, w0_ref, w1_ref, w2_ref, o_ref):
        htb = ht_ref[...]
        h1 = jnp.maximum(
            lax.dot_general(w0_ref[...], htb, (((1,), (0,)), ((), ())),
                            preferred_element_type=jnp.float32), 0.0)
        h2 = jnp.maximum(
            lax.dot_general(w1_ref[...], h1, (((1,), (0,)), ((), ())),
                            preferred_element_type=jnp.float32), 0.0)
        o_ref[...] = lax.dot_general(
            w2_ref[...], h2, (((1,), (0,)), ((), ())),
            preferred_element_type=jnp.float32)

    return pl.pallas_call(
        body,
        grid=(B // _TB,),
        in_specs=[
            pl.BlockSpec((2 * N_LEVELS, _TB), lambda i: (0, i)),
            pl.BlockSpec((HIDDEN, 2 * N_LEVELS), lambda i: (0, 0)),
            pl.BlockSpec((HIDDEN, HIDDEN), lambda i: (0, 0)),
            pl.BlockSpec((OUT_DIM, HIDDEN), lambda i: (0, 0)),
        ],
        out_specs=pl.BlockSpec((OUT_DIM, _TB), lambda i: (0, i)),
        out_shape=jax.ShapeDtypeStruct((OUT_DIM, B), jnp.float32),
    )(ht, W0, W1, W2)


def kernel(xt, tables, W0, W1, W2):
    xq = xt.reshape(B // 128, 128, 4).transpose(0, 2, 1)
    t = xt[0, 3]
    knots = [i / (NUM_BASIS - 1) for i in range(NUM_BASIS)]
    bvals = []
    for j in range(NUM_BASIS):
        bj = 1.0
        for m in range(NUM_BASIS):
            if m != j:
                bj = bj * (t - knots[m]) / (knots[j] - knots[m])
        bvals.append(bj)
    bb = jnp.broadcast_to(
        jnp.stack(bvals).astype(jnp.float32)[:, None],
        (NUM_BASIS, 16)).reshape(NUM_BASIS * 16)
    tabs = tables.reshape(N_LEVELS, _NEB, 128, F).transpose(0, 1, 3, 2)
    tabc = _sc_tabprep(tabs, bb)
    ht = _sc_encode(xq, tabc)
    ot = _tc_mlp(ht, W0, W1, W2)
    return ot.T


# speedup vs baseline: 1.0089x; 1.0089x over previous
import functools

import numpy as np
import jax
import jax.numpy as jnp
from jax import lax
from jax.experimental import pallas as pl
from jax.experimental.pallas import tpu as pltpu
from jax.experimental.pallas import tpu_sc as plsc

N_LEVELS = 8
F = 8
TSIZE = 1 << 18
HMASK = TSIZE - 1
BASE_RES = 32
MAX_RES = 8192
NUM_BASIS = 4
HIDDEN = 64
B = 262144
OUT_DIM = 6
_scale = np.exp2(np.log2(MAX_RES / BASE_RES) / (N_LEVELS - 1))
RES = [int(np.floor(BASE_RES * _scale ** l)) for l in range(N_LEVELS)]
DENSE = [(r + 1) ** 3 <= TSIZE for r in RES]
PRIME1 = int(np.int32(np.uint32(2654435761).astype(np.int32)))
PRIME2 = int(np.int32(np.uint32(805459861).astype(np.int32)))

_NC = 2
_NS = 16
_NW = _NC * _NS
_PB = B // _NW
_C = 128
_NCHUNK = _PB // _C
_G = _C // 16
_NIDX = N_LEVELS * 8

_NEB = TSIZE // 128
_BEB = 8
_PCH = N_LEVELS * _NEB // _NW


def _sc_tabprep(tabs, bb):
    mesh = plsc.VectorSubcoreMesh(core_axis_name="c", subcore_axis_name="s")

    @functools.partial(
        pl.kernel,
        out_type=jax.ShapeDtypeStruct((N_LEVELS, TSIZE, F), jnp.float32),
        mesh=mesh,
        compiler_params=pltpu.CompilerParams(needs_layout_passes=False,
                                             use_tc_tiling_on_sc=False),
        scratch_types=[
            pltpu.VMEM((_BEB, F, 128), jnp.float32),
            pltpu.VMEM((_BEB * 128, F), jnp.float32),
            pltpu.VMEM((NUM_BASIS * 16,), jnp.float32),
            pltpu.SemaphoreType.DMA,
        ],
    )
    def prep(tabs_h, bb_h, out_h, tin, tout, bbv, sem):
        wid = lax.axis_index("s") * _NC + lax.axis_index("c")
        cid0 = wid * _PCH
        lvl = cid0 // _NEB
        eb0 = cid0 % _NEB

        pltpu.sync_copy(bb_h, bbv)

        def batch(bi, carry):
            eb = eb0 + bi * _BEB
            pltpu.sync_copy(tabs_h.at[lvl, pl.ds(eb, _BEB)], tin)
            bvs = [bbv[pl.ds(16 * j, 16)] for j in range(NUM_BASIS)]
            for e in range(_BEB):
                for g in range(8):
                    fv = [tin[e, f, pl.ds(g * 16, 16)] for f in range(F)]
                    y0 = (bvs[0] * fv[0] + bvs[1] * fv[2]
                          + bvs[2] * fv[4] + bvs[3] * fv[6])
                    y1 = (bvs[0] * fv[1] + bvs[1] * fv[3]
                          + bvs[2] * fv[5] + bvs[3] * fv[7])
                    ridx = (jnp.int32(e * 128 + g * 16)
                            + lax.iota(jnp.int32, 16))
                    plsc.store_scatter(
                        tout, [ridx, jnp.full((16,), 0, jnp.int32)], y0)
                    plsc.store_scatter(
                        tout, [ridx, jnp.full((16,), 1, jnp.int32)], y1)
            pltpu.sync_copy(tout, out_h.at[lvl, pl.ds(eb * 128, _BEB * 128)])
            return carry

        lax.fori_loop(0, _PCH // _BEB, batch, 0, unroll=False)

    return prep(tabs, bb)


def _sc_encode(xq, tab):
    mesh = plsc.VectorSubcoreMesh(core_axis_name="c", subcore_axis_name="s")

    @functools.partial(
        pl.kernel,
        out_type=jax.ShapeDtypeStruct((2 * N_LEVELS, B), jnp.float32),
        mesh=mesh,
        compiler_params=pltpu.CompilerParams(needs_layout_passes=False,
                                             use_tc_tiling_on_sc=False),
        scratch_types=[
            pltpu.VMEM((4, 128), jnp.float32),
            pltpu.VMEM((N_LEVELS, 8 * _C), jnp.int32),
            pltpu.VMEM((_NIDX, 128), jnp.float32),
            pltpu.VMEM((2 * 8 * _C, F), jnp.float32),
            pltpu.VMEM((2 * N_LEVELS, _C), jnp.float32),
            pltpu.SemaphoreType.DMA,
            pltpu.SemaphoreType.DMA,
        ],
    )
    def enc(xq_h, tab_h, out_h, xtv, idxb, wb, rows, hb, sem0, sem1):
        wid = lax.axis_index("s") * _NC + lax.axis_index("c")
        tbase = wid * _PB

        def chunk_body(ci, carry):
            cbase = tbase + ci * _C
            pltpu.sync_copy(xq_h.at[wid * (_PB // 128) + ci], xtv)

            def idx_body(g, c2):
                s = g * 16
                x = xtv[0, pl.ds(s, 16)]
                y = xtv[1, pl.ds(s, 16)]
                z = xtv[2, pl.ds(s, 16)]
                for l in range(N_LEVELS):
                    res = RES[l]
                    px = x * np.float32(res)
                    py = y * np.float32(res)
                    pz = z * np.float32(res)
                    ix = px.astype(jnp.int32)
                    iy = py.astype(jnp.int32)
                    iz = pz.astype(jnp.int32)
                    fx = px - ix.astype(jnp.float32)
                    fy = py - iy.astype(jnp.float32)
                    fz = pz - iz.astype(jnp.float32)
                    if DENSE[l]:
                        st = res + 1
                        a0 = ix
                        a1 = a0 + 1
                        b0 = iy * jnp.int32(st)
                        b1 = b0 + jnp.int32(st)
                        c0 = iz * jnp.int32(st * st)
                        c1 = c0 + jnp.int32(st * st)
                        idx8 = [a + bc + cc
                                for a in (a0, a1) for bc in (b0, b1)
                                for cc in (c0, c1)]
                    else:
                        a0 = ix
                        a1 = ix + 1
                        b0 = iy * jnp.int32(PRIME1)
                        b1 = b0 + jnp.int32(PRIME1)
                        c0 = iz * jnp.int32(PRIME2)
                        c1 = c0 + jnp.int32(PRIME2)
                        idx8 = [(a ^ bc ^ cc) & jnp.int32(HMASK)
                                for a in (a0, a1) for bc in (b0, b1)
                                for cc in (c0, c1)]
                    ux = 1.0 - fx
                    uy = 1.0 - fy
                    uz = 1.0 - fz
                    wxy = [ux * uy, ux * fy, fx * uy, fx * fy]
                    for k in range(8):
                        wv = wxy[k >> 1] * (fz if (k & 1) else uz)
                        idxb[l, pl.ds(k * _C + s, 16)] = idx8[k]
                        wb[l * 8 + k, pl.ds(s, 16)] = wv
                return c2

            lax.fori_loop(0, _G, idx_body, 0, unroll=False)

            sems = (sem0, sem1)

            def fire(l):
                pb = l % 2
                cps = []
                for j in range(8):
                    cps.append(pltpu.async_copy(
                        tab_h.at[l].at[idxb.at[l, pl.ds(j * 128, 128)]],
                        rows.at[pl.ds((pb * 8 + j) * 128, 128)], sems[pb]))
                return cps

            def acc_level(l):
                pb = l % 2

                def acc_body(g, c2):
                    s = g * 16
                    acc = [None, None]
                    for k in range(8):
                        wv = wb[l * 8 + k, pl.ds(s, 16)]
                        ridx = (jnp.int32(pb * 8 * _C + k * _C) + s
                                + lax.iota(jnp.int32, 16))
                        for f in range(2):
                            v = plsc.load_gather(
                                rows, [ridx, jnp.full((16,), f, jnp.int32)])
                            vv = wv * v
                            acc[f] = vv if acc[f] is None else acc[f] + vv
                    hb[2 * l, pl.ds(s, 16)] = acc[0]
                    hb[2 * l + 1, pl.ds(s, 16)] = acc[1]
                    return c2

                lax.fori_loop(0, _G, acc_body, 0, unroll=False)

            inflight = fire(0)
            for l in range(N_LEVELS):
                nxt = fire(l + 1) if l + 1 < N_LEVELS else []
                for cp in inflight:
                    cp.wait()
                inflight = nxt
                acc_level(l)

            pltpu.sync_copy(hb, out_h.at[:, pl.ds(cbase, _C)])
            return carry

        lax.fori_loop(0, _NCHUNK, chunk_body, 0, unroll=False)

    return enc(xq, tab)


_TB = 2048


def _tc_mlp(ht, W0, W1, W2):
    def body(ht_ref, w0_ref, w1_ref, w2_ref, o_ref):
        htb = ht_ref[...]
        h1 = jnp.maximum(
            lax.dot_general(w0_ref[...], htb, (((1,), (0,)), ((), ())),
                            preferred_element_type=jnp.float32), 0.0)
        h2 = jnp.maximum(
            lax.dot_general(w1_ref[...], h1, (((1,), (0,)), ((), ())),
                            preferred_element_type=jnp.float32), 0.0)
        o_ref[...] = lax.dot_general(
            w2_ref[...], h2, (((1,), (0,)), ((), ())),
            preferred_element_type=jnp.float32)

    return pl.pallas_call(
        body,
        grid=(B // _TB,),
        in_specs=[
            pl.BlockSpec((2 * N_LEVELS, _TB), lambda i: (0, i)),
            pl.BlockSpec((HIDDEN, 2 * N_LEVELS), lambda i: (0, 0)),
            pl.BlockSpec((HIDDEN, HIDDEN), lambda i: (0, 0)),
            pl.BlockSpec((OUT_DIM, HIDDEN), lambda i: (0, 0)),
        ],
        out_specs=pl.BlockSpec((OUT_DIM, _TB), lambda i: (0, i)),
        out_shape=jax.ShapeDtypeStruct((OUT_DIM, B), jnp.float32),
    )(ht, W0, W1, W2)


def kernel(xt, tables, W0, W1, W2):
    xq = xt.reshape(B // 128, 128, 4).transpose(0, 2, 1)
    t = xt[0, 3]
    knots = [i / (NUM_BASIS - 1) for i in range(NUM_BASIS)]
    bvals = []
    for j in range(NUM_BASIS):
        bj = 1.0
        for m in range(NUM_BASIS):
            if m != j:
                bj = bj * (t - knots[m]) / (knots[j] - knots[m])
        bvals.append(bj)
    bb = jnp.broadcast_to(
        jnp.stack(bvals).astype(jnp.float32)[:, None],
        (NUM_BASIS, 16)).reshape(NUM_BASIS * 16)
    tabs = tables.reshape(N_LEVELS, _NEB, 128, F).transpose(0, 1, 3, 2)
    tabc = _sc_tabprep(tabs, bb)
    ht = _sc_encode(xq, tabc)
    ot = _tc_mlp(ht, W0, W1, W2)
    return ot.T


# speedup vs baseline: 1.0107x; 1.0018x over previous
import functools

import numpy as np
import jax
import jax.numpy as jnp
from jax import lax
from jax.experimental import pallas as pl
from jax.experimental.pallas import tpu as pltpu
from jax.experimental.pallas import tpu_sc as plsc

N_LEVELS = 8
F = 8
TSIZE = 1 << 18
HMASK = TSIZE - 1
BASE_RES = 32
MAX_RES = 8192
NUM_BASIS = 4
HIDDEN = 64
B = 262144
OUT_DIM = 6
_scale = np.exp2(np.log2(MAX_RES / BASE_RES) / (N_LEVELS - 1))
RES = [int(np.floor(BASE_RES * _scale ** l)) for l in range(N_LEVELS)]
DENSE = [(r + 1) ** 3 <= TSIZE for r in RES]
PRIME1 = int(np.int32(np.uint32(2654435761).astype(np.int32)))
PRIME2 = int(np.int32(np.uint32(805459861).astype(np.int32)))

_NC = 2
_NS = 16
_NW = _NC * _NS
_PB = B // _NW
_C = 128
_NCHUNK = _PB // _C
_G = _C // 16
_NIDX = N_LEVELS * 8

_NEB = TSIZE // 128
_BEB = 8
_PCH = N_LEVELS * _NEB // _NW


def _sc_tabprep(tabs, bb):
    mesh = plsc.VectorSubcoreMesh(core_axis_name="c", subcore_axis_name="s")

    @functools.partial(
        pl.kernel,
        out_type=jax.ShapeDtypeStruct((N_LEVELS, TSIZE, F), jnp.float32),
        mesh=mesh,
        compiler_params=pltpu.CompilerParams(needs_layout_passes=False,
                                             use_tc_tiling_on_sc=False),
        scratch_types=[
            pltpu.VMEM((_BEB, F, 128), jnp.float32),
            pltpu.VMEM((_BEB * 128, F), jnp.float32),
            pltpu.VMEM((NUM_BASIS * 16,), jnp.float32),
            pltpu.SemaphoreType.DMA,
        ],
    )
    def prep(tabs_h, bb_h, out_h, tin, tout, bbv, sem):
        wid = lax.axis_index("s") * _NC + lax.axis_index("c")
        cid0 = wid * _PCH
        lvl = cid0 // _NEB
        eb0 = cid0 % _NEB

        pltpu.sync_copy(bb_h, bbv)

        def batch(bi, carry):
            eb = eb0 + bi * _BEB
            pltpu.sync_copy(tabs_h.at[lvl, pl.ds(eb, _BEB)], tin)
            bvs = [bbv[pl.ds(16 * j, 16)] for j in range(NUM_BASIS)]
            for e in range(_BEB):
                for g in range(8):
                    fv = [tin[e, f, pl.ds(g * 16, 16)] for f in range(F)]
                    y0 = (bvs[0] * fv[0] + bvs[1] * fv[2]
                          + bvs[2] * fv[4] + bvs[3] * fv[6])
                    y1 = (bvs[0] * fv[1] + bvs[1] * fv[3]
                          + bvs[2] * fv[5] + bvs[3] * fv[7])
                    ridx = (jnp.int32(e * 128 + g * 16)
                            + lax.iota(jnp.int32, 16))
                    plsc.store_scatter(
                        tout, [ridx, jnp.full((16,), 0, jnp.int32)], y0)
                    plsc.store_scatter(
                        tout, [ridx, jnp.full((16,), 1, jnp.int32)], y1)
            pltpu.sync_copy(tout, out_h.at[lvl, pl.ds(eb * 128, _BEB * 128)])
            return carry

        lax.fori_loop(0, _PCH // _BEB, batch, 0, unroll=False)

    return prep(tabs, bb)


def _sc_encode(xq, tab):
    mesh = plsc.VectorSubcoreMesh(core_axis_name="c", subcore_axis_name="s")

    @functools.partial(
        pl.kernel,
        out_type=jax.ShapeDtypeStruct((2 * N_LEVELS, B), jnp.float32),
        mesh=mesh,
        compiler_params=pltpu.CompilerParams(needs_layout_passes=False,
                                             use_tc_tiling_on_sc=False),
        scratch_types=[
            pltpu.VMEM((4, 128), jnp.float32),
            pltpu.VMEM((_NIDX, 128), jnp.int32),
            pltpu.VMEM((_NIDX, 128), jnp.float32),
            pltpu.VMEM((2 * 8 * _C, F), jnp.float32),
            pltpu.VMEM((2 * N_LEVELS, _C), jnp.float32),
            pltpu.SemaphoreType.DMA,
            pltpu.SemaphoreType.DMA,
        ],
    )
    def enc(xq_h, tab_h, out_h, xtv, idxb, wb, rows, hb, sem0, sem1):
        wid = lax.axis_index("s") * _NC + lax.axis_index("c")
        tbase = wid * _PB

        def chunk_body(ci, carry):
            cbase = tbase + ci * _C
            pltpu.sync_copy(xq_h.at[wid * (_PB // 128) + ci], xtv)

            def idx_body(g, c2):
                s = g * 16
                x = xtv[0, pl.ds(s, 16)]
                y = xtv[1, pl.ds(s, 16)]
                z = xtv[2, pl.ds(s, 16)]
                for l in range(N_LEVELS):
                    res = RES[l]
                    px = x * np.float32(res)
                    py = y * np.float32(res)
                    pz = z * np.float32(res)
                    ix = px.astype(jnp.int32)
                    iy = py.astype(jnp.int32)
                    iz = pz.astype(jnp.int32)
                    fx = px - ix.astype(jnp.float32)
                    fy = py - iy.astype(jnp.float32)
                    fz = pz - iz.astype(jnp.float32)
                    if DENSE[l]:
                        st = res + 1
                        a0 = ix
                        a1 = a0 + 1
                        b0 = iy * jnp.int32(st)
                        b1 = b0 + jnp.int32(st)
                        c0 = iz * jnp.int32(st * st)
                        c1 = c0 + jnp.int32(st * st)
                        idx8 = [a + bc + cc
                                for a in (a0, a1) for bc in (b0, b1)
                                for cc in (c0, c1)]
                    else:
                        a0 = ix
                        a1 = ix + 1
                        b0 = iy * jnp.int32(PRIME1)
                        b1 = b0 + jnp.int32(PRIME1)
                        c0 = iz * jnp.int32(PRIME2)
                        c1 = c0 + jnp.int32(PRIME2)
                        idx8 = [(a ^ bc ^ cc) & jnp.int32(HMASK)
                                for a in (a0, a1) for bc in (b0, b1)
                                for cc in (c0, c1)]
                    ux = 1.0 - fx
                    uy = 1.0 - fy
                    uz = 1.0 - fz
                    wxy = [ux * uy, ux * fy, fx * uy, fx * fy]
                    for k in range(8):
                        wv = wxy[k >> 1] * (fz if (k & 1) else uz)
                        idxb[l * 8 + k, pl.ds(s, 16)] = idx8[k]
                        wb[l * 8 + k, pl.ds(s, 16)] = wv
                return c2

            lax.fori_loop(0, _G, idx_body, 0, unroll=False)

            sems = (sem0, sem1)

            def fire(l):
                pb = l % 2
                cps = []
                for j in range(8):
                    cps.append(pltpu.async_copy(
                        tab_h.at[l].at[idxb.at[l * 8 + j]],
                        rows.at[pl.ds((pb * 8 + j) * 128, 128)], sems[pb]))
                return cps

            def acc_level(l):
                pb = l % 2

                def acc_body(g, c2):
                    s = g * 16
                    acc = [None, None]
                    for k in range(8):
                        wv = wb[l * 8 + k, pl.ds(s, 16)]
                        ridx = (jnp.int32(pb * 8 * _C + k * _C) + s
                                + lax.iota(jnp.int32, 16))
                        for f in range(2):
                            v = plsc.load_gather(
                                rows, [ridx, jnp.full((16,), f, jnp.int32)])
                            vv = wv * v
                            acc[f] = vv if acc[f] is None else acc[f] + vv
                    hb[2 * l, pl.ds(s, 16)] = acc[0]
                    hb[2 * l + 1, pl.ds(s, 16)] = acc[1]
                    return c2

                lax.fori_loop(0, _G, acc_body, 0, unroll=False)

            inflight = fire(0)
            for l in range(N_LEVELS):
                nxt = fire(l + 1) if l + 1 < N_LEVELS else []
                for cp in inflight:
                    cp.wait()
                inflight = nxt
                acc_level(l)

            pltpu.sync_copy(hb, out_h.at[:, pl.ds(cbase, _C)])
            return carry

        lax.fori_loop(0, _NCHUNK, chunk_body, 0, unroll=False)

    return enc(xq, tab)


_TB = 2048


def _tc_mlp(ht, W0, W1, W2):
    def body(ht_ref, w0_ref, w1_ref, w2_ref, o_ref):
        htb = ht_ref[...]
        h1 = jnp.maximum(
            lax.dot_general(w0_ref[...], htb, (((1,), (0,)), ((), ())),
                            preferred_element_type=jnp.float32), 0.0)
        h2 = jnp.maximum(
            lax.dot_general(w1_ref[...], h1, (((1,), (0,)), ((), ())),
                            preferred_element_type=jnp.float32), 0.0)
        o_ref[...] = lax.dot_general(
            w2_ref[...], h2, (((1,), (0,)), ((), ())),
            preferred_element_type=jnp.float32)

    return pl.pallas_call(
        body,
        grid=(B // _TB,),
        in_specs=[
            pl.BlockSpec((2 * N_LEVELS, _TB), lambda i: (0, i)),
            pl.BlockSpec((HIDDEN, 2 * N_LEVELS), lambda i: (0, 0)),
            pl.BlockSpec((HIDDEN, HIDDEN), lambda i: (0, 0)),
            pl.BlockSpec((OUT_DIM, HIDDEN), lambda i: (0, 0)),
        ],
        out_specs=pl.BlockSpec((OUT_DIM, _TB), lambda i: (0, i)),
        out_shape=jax.ShapeDtypeStruct((OUT_DIM, B), jnp.float32),
    )(ht, W0, W1, W2)


def kernel(xt, tables, W0, W1, W2):
    xq = xt.reshape(B // 128, 128, 4).transpose(0, 2, 1)
    t = xt[0, 3]
    knots = [i / (NUM_BASIS - 1) for i in range(NUM_BASIS)]
    bvals = []
    for j in range(NUM_BASIS):
        bj = 1.0
        for m in range(NUM_BASIS):
            if m != j:
                bj = bj * (t - knots[m]) / (knots[j] - knots[m])
        bvals.append(bj)
    bb = jnp.broadcast_to(
        jnp.stack(bvals).astype(jnp.float32)[:, None],
        (NUM_BASIS, 16)).reshape(NUM_BASIS * 16)
    tabs = tables.reshape(N_LEVELS, _NEB, 128, F).transpose(0, 1, 3, 2)
    tabc = _sc_tabprep(tabs, bb)
    ht = _sc_encode(xq, tabc)
    ot = _tc_mlp(ht, W0, W1, W2)
    return ot.T


# speedup vs baseline: 1.0437x; 1.0326x over previous
import functools

import numpy as np
import jax
import jax.numpy as jnp
from jax import lax
from jax.experimental import pallas as pl
from jax.experimental.pallas import tpu as pltpu
from jax.experimental.pallas import tpu_sc as plsc

N_LEVELS = 8
F = 8
TSIZE = 1 << 18
HMASK = TSIZE - 1
BASE_RES = 32
MAX_RES = 8192
NUM_BASIS = 4
HIDDEN = 64
B = 262144
OUT_DIM = 6
_scale = np.exp2(np.log2(MAX_RES / BASE_RES) / (N_LEVELS - 1))
RES = [int(np.floor(BASE_RES * _scale ** l)) for l in range(N_LEVELS)]
DENSE = [(r + 1) ** 3 <= TSIZE for r in RES]
PRIME1 = int(np.int32(np.uint32(2654435761).astype(np.int32)))
PRIME2 = int(np.int32(np.uint32(805459861).astype(np.int32)))

_NC = 2
_NS = 16
_NW = _NC * _NS
_PB = B // _NW
_C = 128
_NCHUNK = _PB // _C
_G = _C // 16
_NIDX = N_LEVELS * 8

_NEB = TSIZE // 128
_BEB = 8
_PCH = N_LEVELS * _NEB // _NW


def _sc_tabprep(tabs, bb):
    mesh = plsc.VectorSubcoreMesh(core_axis_name="c", subcore_axis_name="s")

    @functools.partial(
        pl.kernel,
        out_type=jax.ShapeDtypeStruct((N_LEVELS, TSIZE, F), jnp.float32),
        mesh=mesh,
        compiler_params=pltpu.CompilerParams(needs_layout_passes=False,
                                             use_tc_tiling_on_sc=False),
        scratch_types=[
            pltpu.VMEM((_BEB, F, 128), jnp.float32),
            pltpu.VMEM((_BEB * 128, F), jnp.float32),
            pltpu.VMEM((NUM_BASIS * 16,), jnp.float32),
            pltpu.SemaphoreType.DMA,
        ],
    )
    def prep(tabs_h, bb_h, out_h, tin, tout, bbv, sem):
        wid = lax.axis_index("s") * _NC + lax.axis_index("c")
        cid0 = wid * _PCH
        lvl = cid0 // _NEB
        eb0 = cid0 % _NEB

        pltpu.sync_copy(bb_h, bbv)

        def batch(bi, carry):
            eb = eb0 + bi * _BEB
            pltpu.sync_copy(tabs_h.at[lvl, pl.ds(eb, _BEB)], tin)
            bvs = [bbv[pl.ds(16 * j, 16)] for j in range(NUM_BASIS)]
            for e in range(_BEB):
                for g in range(8):
                    fv = [tin[e, f, pl.ds(g * 16, 16)] for f in range(F)]
                    y0 = (bvs[0] * fv[0] + bvs[1] * fv[2]
                          + bvs[2] * fv[4] + bvs[3] * fv[6])
                    y1 = (bvs[0] * fv[1] + bvs[1] * fv[3]
                          + bvs[2] * fv[5] + bvs[3] * fv[7])
                    ridx = (jnp.int32(e * 128 + g * 16)
                            + lax.iota(jnp.int32, 16))
                    plsc.store_scatter(
                        tout, [ridx, jnp.full((16,), 0, jnp.int32)], y0)
                    plsc.store_scatter(
                        tout, [ridx, jnp.full((16,), 1, jnp.int32)], y1)
            pltpu.sync_copy(tout, out_h.at[lvl, pl.ds(eb * 128, _BEB * 128)])
            return carry

        lax.fori_loop(0, _PCH // _BEB, batch, 0, unroll=False)

    return prep(tabs, bb)


def _sc_encode(xs, tab):
    mesh = plsc.VectorSubcoreMesh(core_axis_name="c", subcore_axis_name="s")

    @functools.partial(
        pl.kernel,
        out_type=jax.ShapeDtypeStruct((2 * N_LEVELS, B), jnp.float32),
        mesh=mesh,
        compiler_params=pltpu.CompilerParams(needs_layout_passes=False,
                                             use_tc_tiling_on_sc=False),
        scratch_types=[
            pltpu.VMEM((4 * _C,), jnp.float32),
            pltpu.VMEM((_NIDX, 128), jnp.int32),
            pltpu.VMEM((_NIDX, 128), jnp.float32),
            pltpu.VMEM((2 * 8 * _C, F), jnp.float32),
            pltpu.VMEM((2 * N_LEVELS, _C), jnp.float32),
            pltpu.SemaphoreType.DMA,
            pltpu.SemaphoreType.DMA,
        ],
    )
    def enc(xs_h, tab_h, out_h, xtv, idxb, wb, rows, hb, sem0, sem1):
        wid = lax.axis_index("s") * _NC + lax.axis_index("c")
        tbase = wid * _PB

        def chunk_body(ci, carry):
            cbase = tbase + ci * _C
            pltpu.sync_copy(xs_h.at[pl.ds(4 * cbase, 4 * _C)], xtv)

            def idx_body(g, c2):
                s = g * 16
                s4 = g * 64
                i4 = s4 + lax.iota(jnp.int32, 16) * 4
                x = plsc.load_gather(xtv, [i4])
                y = plsc.load_gather(xtv, [i4 + 1])
                z = plsc.load_gather(xtv, [i4 + 2])
                for l in range(N_LEVELS):
                    res = RES[l]
                    px = x * np.float32(res)
                    py = y * np.float32(res)
                    pz = z * np.float32(res)
                    ix = px.astype(jnp.int32)
                    iy = py.astype(jnp.int32)
                    iz = pz.astype(jnp.int32)
                    fx = px - ix.astype(jnp.float32)
                    fy = py - iy.astype(jnp.float32)
                    fz = pz - iz.astype(jnp.float32)
                    if DENSE[l]:
                        st = res + 1
                        a0 = ix
                        a1 = a0 + 1
                        b0 = iy * jnp.int32(st)
                        b1 = b0 + jnp.int32(st)
                        c0 = iz * jnp.int32(st * st)
                        c1 = c0 + jnp.int32(st * st)
                        idx8 = [a + bc + cc
                                for a in (a0, a1) for bc in (b0, b1)
                                for cc in (c0, c1)]
                    else:
                        a0 = ix
                        a1 = ix + 1
                        b0 = iy * jnp.int32(PRIME1)
                        b1 = b0 + jnp.int32(PRIME1)
                        c0 = iz * jnp.int32(PRIME2)
                        c1 = c0 + jnp.int32(PRIME2)
                        idx8 = [(a ^ bc ^ cc) & jnp.int32(HMASK)
                                for a in (a0, a1) for bc in (b0, b1)
                                for cc in (c0, c1)]
                    ux = 1.0 - fx
                    uy = 1.0 - fy
                    uz = 1.0 - fz
                    wxy = [ux * uy, ux * fy, fx * uy, fx * fy]
                    for k in range(8):
                        wv = wxy[k >> 1] * (fz if (k & 1) else uz)
                        idxb[l * 8 + k, pl.ds(s, 16)] = idx8[k]
                        wb[l * 8 + k, pl.ds(s, 16)] = wv
                return c2

            lax.fori_loop(0, _G, idx_body, 0, unroll=False)

            sems = (sem0, sem1)

            def fire(l):
                pb = l % 2
                cps = []
                for j in range(8):
                    cps.append(pltpu.async_copy(
                        tab_h.at[l].at[idxb.at[l * 8 + j]],
                        rows.at[pl.ds((pb * 8 + j) * 128, 128)], sems[pb]))
                return cps

            def acc_level(l):
                pb = l % 2

                def acc_body(g, c2):
                    s = g * 16
                    acc = [None, None]
                    for k in range(8):
                        wv = wb[l * 8 + k, pl.ds(s, 16)]
                        ridx = (jnp.int32(pb * 8 * _C + k * _C) + s
                                + lax.iota(jnp.int32, 16))
                        for f in range(2):
                            v = plsc.load_gather(
                                rows, [ridx, jnp.full((16,), f, jnp.int32)])
                            vv = wv * v
                            acc[f] = vv if acc[f] is None else acc[f] + vv
                    hb[2 * l, pl.ds(s, 16)] = acc[0]
                    hb[2 * l + 1, pl.ds(s, 16)] = acc[1]
                    return c2

                lax.fori_loop(0, _G, acc_body, 0, unroll=False)

            inflight = fire(0)
            for l in range(N_LEVELS):
                nxt = fire(l + 1) if l + 1 < N_LEVELS else []
                for cp in inflight:
                    cp.wait()
                inflight = nxt
                acc_level(l)

            pltpu.sync_copy(hb, out_h.at[:, pl.ds(cbase, _C)])
            return carry

        lax.fori_loop(0, _NCHUNK, chunk_body, 0, unroll=False)

    return enc(xs, tab)


_TB = 2048


def _tc_mlp(ht, W0, W1, W2):
    def body(ht_ref, w0_ref, w1_ref, w2_ref, o_ref):
        htb = ht_ref[...]
        h1 = jnp.maximum(
            lax.dot_general(w0_ref[...], htb, (((1,), (0,)), ((), ())),
                            preferred_element_type=jnp.float32), 0.0)
        h2 = jnp.maximum(
            lax.dot_general(w1_ref[...], h1, (((1,), (0,)), ((), ())),
                            preferred_element_type=jnp.float32), 0.0)
        o_ref[...] = lax.dot_general(
            w2_ref[...], h2, (((1,), (0,)), ((), ())),
            preferred_element_type=jnp.float32)

    return pl.pallas_call(
        body,
        grid=(B // _TB,),
        in_specs=[
            pl.BlockSpec((2 * N_LEVELS, _TB), lambda i: (0, i)),
            pl.BlockSpec((HIDDEN, 2 * N_LEVELS), lambda i: (0, 0)),
            pl.BlockSpec((HIDDEN, HIDDEN), lambda i: (0, 0)),
            pl.BlockSpec((OUT_DIM, HIDDEN), lambda i: (0, 0)),
        ],
        out_specs=pl.BlockSpec((OUT_DIM, _TB), lambda i: (0, i)),
        out_shape=jax.ShapeDtypeStruct((OUT_DIM, B), jnp.float32),
    )(ht, W0, W1, W2)


def kernel(xt, tables, W0, W1, W2):
    xs = xt.reshape(4 * B)
    t = xt[0, 3]
    knots = [i / (NUM_BASIS - 1) for i in range(NUM_BASIS)]
    bvals = []
    for j in range(NUM_BASIS):
        bj = 1.0
        for m in range(NUM_BASIS):
            if m != j:
                bj = bj * (t - knots[m]) / (knots[j] - knots[m])
        bvals.append(bj)
    bb = jnp.broadcast_to(
        jnp.stack(bvals).astype(jnp.float32)[:, None],
        (NUM_BASIS, 16)).reshape(NUM_BASIS * 16)
    tabs = tables.reshape(N_LEVELS, _NEB, 128, F).transpose(0, 1, 3, 2)
    tabc = _sc_tabprep(tabs, bb)
    ht = _sc_encode(xs, tabc)
    ot = _tc_mlp(ht, W0, W1, W2)
    return ot.T


# speedup vs baseline: 1.1046x; 1.0584x over previous
import functools

import numpy as np
import jax
import jax.numpy as jnp
from jax import lax
from jax.experimental import pallas as pl
from jax.experimental.pallas import tpu as pltpu
from jax.experimental.pallas import tpu_sc as plsc

N_LEVELS = 8
F = 8
TSIZE = 1 << 18
HMASK = TSIZE - 1
BASE_RES = 32
MAX_RES = 8192
NUM_BASIS = 4
HIDDEN = 64
B = 262144
OUT_DIM = 6
_scale = np.exp2(np.log2(MAX_RES / BASE_RES) / (N_LEVELS - 1))
RES = [int(np.floor(BASE_RES * _scale ** l)) for l in range(N_LEVELS)]
DENSE = [(r + 1) ** 3 <= TSIZE for r in RES]
PRIME1 = int(np.int32(np.uint32(2654435761).astype(np.int32)))
PRIME2 = int(np.int32(np.uint32(805459861).astype(np.int32)))

_NC = 2
_NS = 16
_NW = _NC * _NS
_PB = B // _NW
_C = 128
_NCHUNK = _PB // _C
_G = _C // 16
_NIDX = N_LEVELS * 8

_NEB = TSIZE // 128
_BEB = 8
_PCH = N_LEVELS * _NEB // _NW


def _sc_tabprep(tabs, bb):
    mesh = plsc.VectorSubcoreMesh(core_axis_name="c", subcore_axis_name="s")

    @functools.partial(
        pl.kernel,
        out_type=jax.ShapeDtypeStruct((N_LEVELS, TSIZE, F), jnp.float32),
        mesh=mesh,
        compiler_params=pltpu.CompilerParams(needs_layout_passes=False,
                                             use_tc_tiling_on_sc=False),
        scratch_types=[
            pltpu.VMEM((_BEB, F, 128), jnp.float32),
            pltpu.VMEM((_BEB * 128, F), jnp.float32),
            pltpu.VMEM((NUM_BASIS * 16,), jnp.float32),
            pltpu.SemaphoreType.DMA,
        ],
    )
    def prep(tabs_h, bb_h, out_h, tin, tout, bbv, sem):
        wid = lax.axis_index("s") * _NC + lax.axis_index("c")
        cid0 = wid * _PCH
        lvl = cid0 // _NEB
        eb0 = cid0 % _NEB

        pltpu.sync_copy(bb_h, bbv)

        def batch(bi, carry):
            eb = eb0 + bi * _BEB
            pltpu.sync_copy(tabs_h.at[lvl, pl.ds(eb, _BEB)], tin)
            bvs = [bbv[pl.ds(16 * j, 16)] for j in range(NUM_BASIS)]
            for e in range(_BEB):
                for g in range(8):
                    fv = [tin[e, f, pl.ds(g * 16, 16)] for f in range(F)]
                    y0 = (bvs[0] * fv[0] + bvs[1] * fv[2]
                          + bvs[2] * fv[4] + bvs[3] * fv[6])
                    y1 = (bvs[0] * fv[1] + bvs[1] * fv[3]
                          + bvs[2] * fv[5] + bvs[3] * fv[7])
                    ridx = (jnp.int32(e * 128 + g * 16)
                            + lax.iota(jnp.int32, 16))
                    plsc.store_scatter(
                        tout, [ridx, jnp.full((16,), 0, jnp.int32)], y0)
                    plsc.store_scatter(
                        tout, [ridx, jnp.full((16,), 1, jnp.int32)], y1)
            pltpu.sync_copy(tout, out_h.at[lvl, pl.ds(eb * 128, _BEB * 128)])
            return carry

        lax.fori_loop(0, _PCH // _BEB, batch, 0, unroll=False)

    return prep(tabs, bb)


def _sc_encode(xs, tab):
    mesh = plsc.VectorSubcoreMesh(core_axis_name="c", subcore_axis_name="s")

    @functools.partial(
        pl.kernel,
        out_type=jax.ShapeDtypeStruct((2 * N_LEVELS, B), jnp.float32),
        mesh=mesh,
        compiler_params=pltpu.CompilerParams(needs_layout_passes=False,
                                             use_tc_tiling_on_sc=False),
        scratch_types=[
            pltpu.VMEM((4 * _C,), jnp.float32),
            pltpu.VMEM((_NIDX, 128), jnp.int32),
            pltpu.VMEM((_NIDX, 128), jnp.float32),
            pltpu.VMEM((2 * 8 * _C, F), jnp.float32),
            pltpu.VMEM((2 * N_LEVELS, _C), jnp.float32),
            pltpu.SemaphoreType.DMA,
            pltpu.SemaphoreType.DMA,
        ],
    )
    def enc(xs_h, tab_h, out_h, xtv, idxb, wb, rows, hb, sem0, sem1):
        wid = lax.axis_index("s") * _NC + lax.axis_index("c")
        tbase = wid * _PB

        def chunk_body(ci, carry):
            cbase = tbase + ci * _C
            pltpu.sync_copy(xs_h.at[pl.ds(4 * cbase, 4 * _C)], xtv)

            def idx_body(g, c2):
                s = g * 16
                s4 = g * 64
                i4 = s4 + lax.iota(jnp.int32, 16) * 4
                x = plsc.load_gather(xtv, [i4])
                y = plsc.load_gather(xtv, [i4 + 1])
                z = plsc.load_gather(xtv, [i4 + 2])
                for l in range(N_LEVELS):
                    res = RES[l]
                    px = x * np.float32(res)
                    py = y * np.float32(res)
                    pz = z * np.float32(res)
                    ix = px.astype(jnp.int32)
                    iy = py.astype(jnp.int32)
                    iz = pz.astype(jnp.int32)
                    fx = px - ix.astype(jnp.float32)
                    fy = py - iy.astype(jnp.float32)
                    fz = pz - iz.astype(jnp.float32)
                    if DENSE[l]:
                        st = res + 1
                        a0 = ix
                        a1 = a0 + 1
                        b0 = iy * jnp.int32(st)
                        b1 = b0 + jnp.int32(st)
                        c0 = iz * jnp.int32(st * st)
                        c1 = c0 + jnp.int32(st * st)
                        idx8 = [a + bc + cc
                                for a in (a0, a1) for bc in (b0, b1)
                                for cc in (c0, c1)]
                    else:
                        a0 = ix
                        a1 = ix + 1
                        b0 = iy * jnp.int32(PRIME1)
                        b1 = b0 + jnp.int32(PRIME1)
                        c0 = iz * jnp.int32(PRIME2)
                        c1 = c0 + jnp.int32(PRIME2)
                        idx8 = [(a ^ bc ^ cc) & jnp.int32(HMASK)
                                for a in (a0, a1) for bc in (b0, b1)
                                for cc in (c0, c1)]
                    ux = 1.0 - fx
                    uy = 1.0 - fy
                    uz = 1.0 - fz
                    wxy = [ux * uy, ux * fy, fx * uy, fx * fy]
                    for k in range(8):
                        wv = wxy[k >> 1] * (fz if (k & 1) else uz)
                        idxb[l * 8 + k, pl.ds(s, 16)] = idx8[k]
                        wb[l * 8 + k, pl.ds(s, 16)] = wv
                return c2

            lax.fori_loop(0, _G, idx_body, 0, unroll=False)

            sems = (sem0, sem1)

            def fire(l):
                pb = l % 2
                cps = []
                for j in range(8):
                    cps.append(pltpu.async_copy(
                        tab_h.at[l].at[idxb.at[l * 8 + j]],
                        rows.at[pl.ds((pb * 8 + j) * 128, 128)], sems[pb]))
                return cps

            def acc_level(l):
                pb = l % 2

                def acc_body(g, c2):
                    s = g * 16
                    acc = [None, None]
                    for k in range(8):
                        wv = wb[l * 8 + k, pl.ds(s, 16)]
                        ridx = (jnp.int32(pb * 8 * _C + k * _C) + s
                                + lax.iota(jnp.int32, 16))
                        for f in range(2):
                            v = plsc.load_gather(
                                rows, [ridx, jnp.full((16,), f, jnp.int32)])
                            vv = wv * v
                            acc[f] = vv if acc[f] is None else acc[f] + vv
                    hb[2 * l, pl.ds(s, 16)] = acc[0]
                    hb[2 * l + 1, pl.ds(s, 16)] = acc[1]
                    return c2

                lax.fori_loop(0, _G, acc_body, 0, unroll=False)

            inflight = fire(0)
            for l in range(N_LEVELS):
                nxt = fire(l + 1) if l + 1 < N_LEVELS else []
                for cp in inflight:
                    cp.wait()
                inflight = nxt
                acc_level(l)

            pltpu.sync_copy(hb, out_h.at[:, pl.ds(cbase, _C)])
            return carry

        lax.fori_loop(0, _NCHUNK, chunk_body, 0, unroll=False)

    return enc(xs, tab)


_TB = 8192


def _tc_mlp(ht, W0, W1, W2):
    def body(ht_ref, w0_ref, w1_ref, w2_ref, o_ref):
        htb = ht_ref[...]
        h1 = jnp.maximum(
            lax.dot_general(w0_ref[...], htb, (((1,), (0,)), ((), ())),
                            preferred_element_type=jnp.float32), 0.0)
        h2 = jnp.maximum(
            lax.dot_general(w1_ref[...].astype(jnp.bfloat16),
                            h1.astype(jnp.bfloat16), (((1,), (0,)), ((), ())),
                            preferred_element_type=jnp.float32), 0.0)
        o_ref[...] = lax.dot_general(
            w2_ref[...], h2, (((1,), (0,)), ((), ())),
            preferred_element_type=jnp.float32)

    return pl.pallas_call(
        body,
        grid=(B // _TB,),
        in_specs=[
            pl.BlockSpec((2 * N_LEVELS, _TB), lambda i: (0, i)),
            pl.BlockSpec((HIDDEN, 2 * N_LEVELS), lambda i: (0, 0)),
            pl.BlockSpec((HIDDEN, HIDDEN), lambda i: (0, 0)),
            pl.BlockSpec((OUT_DIM, HIDDEN), lambda i: (0, 0)),
        ],
        out_specs=pl.BlockSpec((OUT_DIM, _TB), lambda i: (0, i)),
        out_shape=jax.ShapeDtypeStruct((OUT_DIM, B), jnp.float32),
    )(ht, W0, W1, W2)


def kernel(xt, tables, W0, W1, W2):
    xs = xt.reshape(4 * B)
    t = xt[0, 3]
    knots = [i / (NUM_BASIS - 1) for i in range(NUM_BASIS)]
    bvals = []
    for j in range(NUM_BASIS):
        bj = 1.0
        for m in range(NUM_BASIS):
            if m != j:
                bj = bj * (t - knots[m]) / (knots[j] - knots[m])
        bvals.append(bj)
    bb = jnp.broadcast_to(
        jnp.stack(bvals).astype(jnp.float32)[:, None],
        (NUM_BASIS, 16)).reshape(NUM_BASIS * 16)
    tabs = tables.reshape(N_LEVELS, _NEB, 128, F).transpose(0, 1, 3, 2)
    tabc = _sc_tabprep(tabs, bb)
    ht = _sc_encode(xs, tabc)
    ot = _tc_mlp(ht, W0, W1, W2)
    return ot.T
